# baseline (speedup 1.0000x reference)
import jax
import jax.numpy as jnp
from jax.experimental import pallas as pl
from jax.experimental.pallas import tpu as pltpu

HIDDEN = 2048
NQ = 16
NKV = 4
DH = 128
G = NQ // NKV
KERNEL_W = 32
STRIDE = 16
WIN = 512
THETA = 500000.0
T = 2048
NUM_C = (T - KERNEL_W) // STRIDE + 1
C_PAD = 128
QT = 256
WCOLS = 3 * QT


def _llama3_inv_freq():
    inv = 1.0 / (THETA ** (jnp.arange(0, DH, 2, dtype=jnp.float32) / DH))
    factor, lo, hi, orig = 8.0, 1.0, 4.0, 8192.0
    wavelen = 2.0 * jnp.pi / inv
    smooth = jnp.clip((orig / wavelen - lo) / (hi - lo), 0.0, 1.0)
    return jnp.where(
        wavelen > orig / lo,
        inv / factor,
        jnp.where(wavelen < orig / hi, inv, (1.0 - smooth) * inv / factor + smooth * inv),
    )


def _prep_body(x_ref, wq_ref, wk_ref, wv_ref, wg_ref, cos_ref, sin_ref,
               q_ref, k_ref, v_ref, g_ref):
    xb = x_ref[...].astype(jnp.bfloat16)
    qp = jnp.dot(xb, wq_ref[...], preferred_element_type=jnp.float32)
    kp = jnp.dot(xb, wk_ref[...], preferred_element_type=jnp.float32)
    vp = jnp.dot(xb, wv_ref[...], preferred_element_type=jnp.float32)
    gp = jnp.dot(xb, wg_ref[...], preferred_element_type=jnp.float32)
    g_ref[...] = jax.nn.sigmoid(gp)
    cos = cos_ref[...]
    sin = sin_ref[...]
    scale = DH ** -0.5
    for h in range(NQ):
        x1 = qp[:, h * DH : h * DH + 64]
        x2 = qp[:, h * DH + 64 : (h + 1) * DH]
        r = jnp.concatenate([x1 * cos - x2 * sin, x1 * sin + x2 * cos], axis=1)
        q_ref[h] = (r * scale).astype(jnp.bfloat16)
    for n in range(NKV):
        x1 = kp[:, n * DH : n * DH + 64]
        x2 = kp[:, n * DH + 64 : (n + 1) * DH]
        r = jnp.concatenate([x1 * cos - x2 * sin, x1 * sin + x2 * cos], axis=1)
        k_ref[n] = r.astype(jnp.bfloat16)
        v_ref[n] = vp[:, n * DH : (n + 1) * DH].astype(jnp.bfloat16)


def _prep(x, wq_p, wk_p, wv, wg_pad, cos, sin):
    return pl.pallas_call(
        _prep_body,
        grid=(T // QT,),
        in_specs=[
            pl.BlockSpec((QT, HIDDEN), lambda i: (i, 0)),
            pl.BlockSpec((HIDDEN, NQ * DH), lambda i: (0, 0)),
            pl.BlockSpec((HIDDEN, NKV * DH), lambda i: (0, 0)),
            pl.BlockSpec((HIDDEN, NKV * DH), lambda i: (0, 0)),
            pl.BlockSpec((HIDDEN, 128), lambda i: (0, 0)),
            pl.BlockSpec((QT, 64), lambda i: (i, 0)),
            pl.BlockSpec((QT, 64), lambda i: (i, 0)),
        ],
        out_specs=[
            pl.BlockSpec((NQ, QT, DH), lambda i: (0, i, 0)),
            pl.BlockSpec((NKV, QT, DH), lambda i: (0, i, 0)),
            pl.BlockSpec((NKV, QT, DH), lambda i: (0, i, 0)),
            pl.BlockSpec((QT, 128), lambda i: (i, 0)),
        ],
        out_shape=[
            jax.ShapeDtypeStruct((NQ, T, DH), jnp.bfloat16),
            jax.ShapeDtypeStruct((NKV, T, DH), jnp.bfloat16),
            jax.ShapeDtypeStruct((NKV, T, DH), jnp.bfloat16),
            jax.ShapeDtypeStruct((T, 128), jnp.float32),
        ],
    )(x, wq_p, wk_p, wv, wg_pad, cos, sin)


def _attn_body(q_ref, k_ref, v_ref, ck_ref, cv_ref, g_ref, o_ref):
    i = pl.program_id(1)
    qb = q_ref[0]
    riota = jax.lax.broadcasted_iota(jnp.int32, (QT, QT), 0) + i * QT
    ciota = jax.lax.broadcasted_iota(jnp.int32, (QT, QT), 1)

    def _scores(j):
        kj = k_ref[0, pl.ds(j * QT, QT), :]
        sj = jax.lax.dot_general(
            qb, kj, (((1,), (1,)), ((), ())), preferred_element_type=jnp.float32
        )
        return sj, ciota + j * QT

    def _p1(j, mx):
        sj, cols = _scores(j)
        sj = jnp.where(riota >= cols, sj, jnp.float32(-1e9))
        return jnp.maximum(mx, jnp.max(sj, axis=-1, keepdims=True))

    mx = jax.lax.fori_loop(0, i + 1, _p1, jnp.full((QT, 1), -1e9, jnp.float32))

    def _p2(j, carry):
        acc_s, l_s = carry
        sj, cols = _scores(j)
        e = jnp.where(riota >= cols, jnp.exp(sj - mx), jnp.float32(0.0))
        vj = v_ref[0, pl.ds(j * QT, QT), :]
        return (
            acc_s + jnp.dot(e.astype(jnp.bfloat16), vj, preferred_element_type=jnp.float32),
            l_s + jnp.sum(e, axis=-1, keepdims=True),
        )

    zero_acc = (jnp.zeros((QT, DH), jnp.float32), jnp.zeros((QT, 1), jnp.float32))
    acc_s, l_s = jax.lax.fori_loop(0, jnp.maximum(i - 2, 0), _p2, zero_acc)

    acc_w = jnp.zeros((QT, DH), jnp.float32)
    l_w = jnp.zeros((QT, 1), jnp.float32)
    for d in range(3):
        jj = i - 2 + d
        jc = jnp.maximum(jj, 0)
        sj, cols = _scores(jc)
        ok = (riota >= cols) & (jj >= 0)
        e = jnp.where(ok, jnp.exp(sj - mx), jnp.float32(0.0))
        vj = v_ref[0, pl.ds(jc * QT, QT), :]
        acc_s = acc_s + jnp.dot(e.astype(jnp.bfloat16), vj, preferred_element_type=jnp.float32)
        l_s = l_s + jnp.sum(e, axis=-1, keepdims=True)
        ew = jnp.where((riota - cols) < WIN, e, jnp.float32(0.0))
        acc_w = acc_w + jnp.dot(ew.astype(jnp.bfloat16), vj, preferred_element_type=jnp.float32)
        l_w = l_w + jnp.sum(ew, axis=-1, keepdims=True)
    out_s = acc_s / l_s
    out_w = acc_w / l_w

    ccols = jax.lax.broadcasted_iota(jnp.int32, (QT, C_PAD), 1)
    crows = jax.lax.broadcasted_iota(jnp.int32, (QT, C_PAD), 0) + i * QT
    cmask = (crows >= ccols * STRIDE + KERNEL_W - 1) & (ccols < NUM_C)
    s_c = jax.lax.dot_general(
        qb, ck_ref[0], (((1,), (1,)), ((), ())), preferred_element_type=jnp.float32
    )
    s_c = jnp.where(cmask, s_c, jnp.float32(-1e9))
    mc = jnp.max(s_c, axis=-1, keepdims=True)
    ec = jnp.exp(s_c - mc)
    l_c = jnp.sum(ec, axis=-1, keepdims=True)
    valid = (crows[:, :1] >= (KERNEL_W - 1)).astype(jnp.float32)
    out_c = jnp.dot(ec.astype(jnp.bfloat16), cv_ref[0], preferred_element_type=jnp.float32)
    out_c = out_c * (valid / l_c)

    g0 = g_ref[0, 0, :][:, None]
    g1 = g_ref[0, 1, :][:, None]
    g2 = g_ref[0, 2, :][:, None]
    o_ref[...] = (g0 * out_c + g1 * out_s + g2 * out_w).astype(jnp.bfloat16)


def _attention(q, k, v, ck, cv, g):
    return pl.pallas_call(
        _attn_body,
        grid=(NQ, T // QT),
        in_specs=[
            pl.BlockSpec((1, QT, DH), lambda h, i: (h, i, 0)),
            pl.BlockSpec((1, T, DH), lambda h, i: (h // G, 0, 0)),
            pl.BlockSpec((1, T, DH), lambda h, i: (h // G, 0, 0)),
            pl.BlockSpec((1, C_PAD, DH), lambda h, i: (h // G, 0, 0)),
            pl.BlockSpec((1, C_PAD, DH), lambda h, i: (h // G, 0, 0)),
            pl.BlockSpec((1, 8, QT), lambda h, i: (h, 0, i)),
        ],
        out_specs=pl.BlockSpec((QT, DH), lambda h, i: (i, h)),
        out_shape=jax.ShapeDtypeStruct((T, NQ * DH), jnp.bfloat16),
    )(q, k, v, ck, cv, g)


def _pool_body(pk_ref, pv_ref, k_ref, v_ref, ck_ref, cv_ref):
    ck_ref[0] = jnp.dot(pk_ref[...], k_ref[0], preferred_element_type=jnp.float32).astype(
        jnp.bfloat16
    )
    cv_ref[0] = jnp.dot(pv_ref[...], v_ref[0], preferred_element_type=jnp.float32).astype(
        jnp.bfloat16
    )


def _pool(pool_k, pool_v, k, v):
    return pl.pallas_call(
        _pool_body,
        grid=(NKV,),
        in_specs=[
            pl.BlockSpec((C_PAD, T), lambda n: (0, 0)),
            pl.BlockSpec((C_PAD, T), lambda n: (0, 0)),
            pl.BlockSpec((1, T, DH), lambda n: (n, 0, 0)),
            pl.BlockSpec((1, T, DH), lambda n: (n, 0, 0)),
        ],
        out_specs=[
            pl.BlockSpec((1, C_PAD, DH), lambda n: (n, 0, 0)),
            pl.BlockSpec((1, C_PAD, DH), lambda n: (n, 0, 0)),
        ],
        out_shape=[
            jax.ShapeDtypeStruct((NKV, C_PAD, DH), jnp.bfloat16),
            jax.ShapeDtypeStruct((NKV, C_PAD, DH), jnp.bfloat16),
        ],
    )(pool_k, pool_v, k, v)


def _mm_body(x_ref, w_ref, o_ref):
    o_ref[...] = jnp.dot(x_ref[...], w_ref[...], preferred_element_type=jnp.float32)


def _matmul(x, w, bn):
    M, K = x.shape
    _, N = w.shape
    return pl.pallas_call(
        _mm_body,
        grid=(N // bn,),
        in_specs=[
            pl.BlockSpec((M, K), lambda j: (0, 0)),
            pl.BlockSpec((K, bn), lambda j: (0, j)),
        ],
        out_specs=pl.BlockSpec((M, bn), lambda j: (0, j)),
        out_shape=jax.ShapeDtypeStruct((M, N), jnp.float32),
    )(x, w)


def _deinterleave_cols(w, nheads):
    return w.reshape(HIDDEN, nheads, 64, 2).transpose(0, 1, 3, 2).reshape(HIDDEN, nheads * DH)


def kernel(hidden_states, Wq, Wk, Wv, Wo, Wg, w_ck, w_cv):
    B, S, H = hidden_states.shape
    x = hidden_states.reshape(B * S, H)

    wq_p = _deinterleave_cols(Wq, NQ).astype(jnp.bfloat16)
    wk_p = _deinterleave_cols(Wk, NKV).astype(jnp.bfloat16)
    wv_b = Wv.astype(jnp.bfloat16)
    wg_pad = jnp.pad(Wg, ((0, 0), (0, 128 - NQ * 3))).astype(jnp.bfloat16)

    pos = jnp.arange(T, dtype=jnp.float32)
    f = pos[:, None] * _llama3_inv_freq()[None, :]
    cos = jnp.cos(f)
    sin = jnp.sin(f)

    qh, kh, vh, gsig = _prep(x, wq_p, wk_p, wv_b, wg_pad, cos, sin)

    wk_c = jax.nn.softmax(w_ck)
    wv_c = jax.nn.softmax(w_cv)
    cpos = jnp.arange(C_PAD)[:, None]
    tpos = jnp.arange(T)[None, :]
    dlt = tpos - cpos * STRIDE
    band = (dlt >= 0) & (dlt < KERNEL_W) & (cpos < NUM_C)
    pool_k = jnp.where(band, jnp.take(wk_c, jnp.clip(dlt, 0, KERNEL_W - 1)), 0.0)
    pool_v = jnp.where(band, jnp.take(wv_c, jnp.clip(dlt, 0, KERNEL_W - 1)), 0.0)
    ck, cv = _pool(pool_k.astype(jnp.bfloat16), pool_v.astype(jnp.bfloat16), kh, vh)

    g_pad = jnp.pad(
        gsig[:, : NQ * 3].reshape(T, NQ, 3).transpose(1, 2, 0), ((0, 0), (0, 5), (0, 0))
    )

    out = _attention(qh, kh, vh, ck, cv, g_pad)
    y = _matmul(out, Wo.astype(jnp.bfloat16), bn=256)
    return y.reshape(B, S, H)


# speedup vs baseline: 1.0558x; 1.0558x over previous
import jax
import jax.numpy as jnp
from jax.experimental import pallas as pl
from jax.experimental.pallas import tpu as pltpu

HIDDEN = 2048
NQ = 16
NKV = 4
DH = 128
G = NQ // NKV
KERNEL_W = 32
STRIDE = 16
WIN = 512
THETA = 500000.0
T = 2048
NUM_C = (T - KERNEL_W) // STRIDE + 1
C_PAD = 128
QT = 256
WCOLS = 3 * QT


def _llama3_inv_freq():
    inv = 1.0 / (THETA ** (jnp.arange(0, DH, 2, dtype=jnp.float32) / DH))
    factor, lo, hi, orig = 8.0, 1.0, 4.0, 8192.0
    wavelen = 2.0 * jnp.pi / inv
    smooth = jnp.clip((orig / wavelen - lo) / (hi - lo), 0.0, 1.0)
    return jnp.where(
        wavelen > orig / lo,
        inv / factor,
        jnp.where(wavelen < orig / hi, inv, (1.0 - smooth) * inv / factor + smooth * inv),
    )


def _prep_body(x_ref, wq_ref, wk_ref, wv_ref, wg_ref, cos_ref, sin_ref,
               q_ref, k_ref, v_ref, g_ref):
    xb = x_ref[...].astype(jnp.bfloat16)
    qp = jnp.dot(xb, wq_ref[...], preferred_element_type=jnp.float32)
    kp = jnp.dot(xb, wk_ref[...], preferred_element_type=jnp.float32)
    vp = jnp.dot(xb, wv_ref[...], preferred_element_type=jnp.float32)
    gp = jnp.dot(xb, wg_ref[...], preferred_element_type=jnp.float32)
    g_ref[...] = jax.nn.sigmoid(gp)
    cos = cos_ref[...]
    sin = sin_ref[...]
    scale = DH ** -0.5
    for h in range(NQ):
        x1 = qp[:, h * DH : h * DH + 64]
        x2 = qp[:, h * DH + 64 : (h + 1) * DH]
        r = jnp.concatenate([x1 * cos - x2 * sin, x1 * sin + x2 * cos], axis=1)
        q_ref[h] = (r * scale).astype(jnp.bfloat16)
    for n in range(NKV):
        x1 = kp[:, n * DH : n * DH + 64]
        x2 = kp[:, n * DH + 64 : (n + 1) * DH]
        r = jnp.concatenate([x1 * cos - x2 * sin, x1 * sin + x2 * cos], axis=1)
        k_ref[n] = r.astype(jnp.bfloat16)
        v_ref[n] = vp[:, n * DH : (n + 1) * DH].astype(jnp.bfloat16)


def _prep(x, wq_p, wk_p, wv, wg_pad, cos, sin):
    return pl.pallas_call(
        _prep_body,
        grid=(T // QT,),
        in_specs=[
            pl.BlockSpec((QT, HIDDEN), lambda i: (i, 0)),
            pl.BlockSpec((HIDDEN, NQ * DH), lambda i: (0, 0)),
            pl.BlockSpec((HIDDEN, NKV * DH), lambda i: (0, 0)),
            pl.BlockSpec((HIDDEN, NKV * DH), lambda i: (0, 0)),
            pl.BlockSpec((HIDDEN, 128), lambda i: (0, 0)),
            pl.BlockSpec((QT, 64), lambda i: (i, 0)),
            pl.BlockSpec((QT, 64), lambda i: (i, 0)),
        ],
        out_specs=[
            pl.BlockSpec((NQ, QT, DH), lambda i: (0, i, 0)),
            pl.BlockSpec((NKV, QT, DH), lambda i: (0, i, 0)),
            pl.BlockSpec((NKV, QT, DH), lambda i: (0, i, 0)),
            pl.BlockSpec((QT, 128), lambda i: (i, 0)),
        ],
        out_shape=[
            jax.ShapeDtypeStruct((NQ, T, DH), jnp.bfloat16),
            jax.ShapeDtypeStruct((NKV, T, DH), jnp.bfloat16),
            jax.ShapeDtypeStruct((NKV, T, DH), jnp.bfloat16),
            jax.ShapeDtypeStruct((T, 128), jnp.float32),
        ],
    )(x, wq_p, wk_p, wv, wg_pad, cos, sin)


NT = T // QT


def _attn_body_for(I):
    KW = (I + 1) * QT
    WS = max(I - 2, 0) * QT
    WC = KW - WS

    def body(q_ref, k_ref, v_ref, ck_ref, cv_ref, g_ref, o_ref):
        qb = q_ref[0]
        kb = k_ref[0]
        rows = jax.lax.broadcasted_iota(jnp.int32, (QT, KW), 0) + I * QT
        cols = jax.lax.broadcasted_iota(jnp.int32, (QT, KW), 1)

        s = jax.lax.dot_general(
            qb, kb, (((1,), (1,)), ((), ())), preferred_element_type=jnp.float32
        )
        s = jnp.where(rows >= cols, s, jnp.float32(-1e9))
        mx = jnp.max(s, axis=-1, keepdims=True)
        e = jnp.exp(s - mx)
        l_s = jnp.sum(e, axis=-1, keepdims=True)
        out_s = jnp.dot(
            e.astype(jnp.bfloat16), v_ref[0], preferred_element_type=jnp.float32
        ) / l_s

        ew = e[:, WS:]
        wrows = rows[:, WS:]
        wcols = cols[:, WS:]
        ew = jnp.where((wrows - wcols) < WIN, ew, jnp.float32(0.0))
        l_w = jnp.sum(ew, axis=-1, keepdims=True)
        vw = v_ref[0, WS:KW, :]
        out_w = jnp.dot(
            ew.astype(jnp.bfloat16), vw, preferred_element_type=jnp.float32
        ) / l_w

        ccols = jax.lax.broadcasted_iota(jnp.int32, (QT, C_PAD), 1)
        crows = jax.lax.broadcasted_iota(jnp.int32, (QT, C_PAD), 0) + I * QT
        cmask = (crows >= ccols * STRIDE + KERNEL_W - 1) & (ccols < NUM_C)
        s_c = jax.lax.dot_general(
            qb, ck_ref[0], (((1,), (1,)), ((), ())), preferred_element_type=jnp.float32
        )
        s_c = jnp.where(cmask, s_c, jnp.float32(-1e9))
        mc = jnp.max(s_c, axis=-1, keepdims=True)
        ec = jnp.exp(s_c - mc)
        l_c = jnp.sum(ec, axis=-1, keepdims=True)
        out_c = jnp.dot(
            ec.astype(jnp.bfloat16), cv_ref[0], preferred_element_type=jnp.float32
        )
        if I == 0:
            valid = (crows[:, :1] >= (KERNEL_W - 1)).astype(jnp.float32)
            out_c = out_c * (valid / l_c)
        else:
            out_c = out_c / l_c

        g0 = g_ref[0, 0, :][:, None]
        g1 = g_ref[0, 1, :][:, None]
        g2 = g_ref[0, 2, :][:, None]
        o_ref[...] = (g0 * out_c + g1 * out_s + g2 * out_w).astype(jnp.bfloat16)

    return body


def _attention(q, k, v, ck, cv, g):
    outs = []
    for I in range(NT):
        KW = (I + 1) * QT
        outs.append(
            pl.pallas_call(
                _attn_body_for(I),
                grid=(NQ,),
                in_specs=[
                    pl.BlockSpec((1, QT, DH), lambda h, I=I: (h, I, 0)),
                    pl.BlockSpec((1, KW, DH), lambda h: (h // G, 0, 0)),
                    pl.BlockSpec((1, KW, DH), lambda h: (h // G, 0, 0)),
                    pl.BlockSpec((1, C_PAD, DH), lambda h: (h // G, 0, 0)),
                    pl.BlockSpec((1, C_PAD, DH), lambda h: (h // G, 0, 0)),
                    pl.BlockSpec((1, 8, QT), lambda h, I=I: (h, 0, I)),
                ],
                out_specs=pl.BlockSpec((QT, DH), lambda h: (0, h)),
                out_shape=jax.ShapeDtypeStruct((QT, NQ * DH), jnp.bfloat16),
            )(q, k, v, ck, cv, g)
        )
    return jnp.concatenate(outs, axis=0)


def _pool_body(pk_ref, pv_ref, k_ref, v_ref, ck_ref, cv_ref):
    ck_ref[0] = jnp.dot(pk_ref[...], k_ref[0], preferred_element_type=jnp.float32).astype(
        jnp.bfloat16
    )
    cv_ref[0] = jnp.dot(pv_ref[...], v_ref[0], preferred_element_type=jnp.float32).astype(
        jnp.bfloat16
    )


def _pool(pool_k, pool_v, k, v):
    return pl.pallas_call(
        _pool_body,
        grid=(NKV,),
        in_specs=[
            pl.BlockSpec((C_PAD, T), lambda n: (0, 0)),
            pl.BlockSpec((C_PAD, T), lambda n: (0, 0)),
            pl.BlockSpec((1, T, DH), lambda n: (n, 0, 0)),
            pl.BlockSpec((1, T, DH), lambda n: (n, 0, 0)),
        ],
        out_specs=[
            pl.BlockSpec((1, C_PAD, DH), lambda n: (n, 0, 0)),
            pl.BlockSpec((1, C_PAD, DH), lambda n: (n, 0, 0)),
        ],
        out_shape=[
            jax.ShapeDtypeStruct((NKV, C_PAD, DH), jnp.bfloat16),
            jax.ShapeDtypeStruct((NKV, C_PAD, DH), jnp.bfloat16),
        ],
    )(pool_k, pool_v, k, v)


def _mm_body(x_ref, w_ref, o_ref):
    o_ref[...] = jnp.dot(x_ref[...], w_ref[...], preferred_element_type=jnp.float32)


def _matmul(x, w, bn):
    M, K = x.shape
    _, N = w.shape
    return pl.pallas_call(
        _mm_body,
        grid=(N // bn,),
        in_specs=[
            pl.BlockSpec((M, K), lambda j: (0, 0)),
            pl.BlockSpec((K, bn), lambda j: (0, j)),
        ],
        out_specs=pl.BlockSpec((M, bn), lambda j: (0, j)),
        out_shape=jax.ShapeDtypeStruct((M, N), jnp.float32),
    )(x, w)


def _deinterleave_cols(w, nheads):
    return w.reshape(HIDDEN, nheads, 64, 2).transpose(0, 1, 3, 2).reshape(HIDDEN, nheads * DH)


def kernel(hidden_states, Wq, Wk, Wv, Wo, Wg, w_ck, w_cv):
    B, S, H = hidden_states.shape
    x = hidden_states.reshape(B * S, H)

    wq_p = _deinterleave_cols(Wq, NQ).astype(jnp.bfloat16)
    wk_p = _deinterleave_cols(Wk, NKV).astype(jnp.bfloat16)
    wv_b = Wv.astype(jnp.bfloat16)
    wg_pad = jnp.pad(Wg, ((0, 0), (0, 128 - NQ * 3))).astype(jnp.bfloat16)

    pos = jnp.arange(T, dtype=jnp.float32)
    f = pos[:, None] * _llama3_inv_freq()[None, :]
    cos = jnp.cos(f)
    sin = jnp.sin(f)

    qh, kh, vh, gsig = _prep(x, wq_p, wk_p, wv_b, wg_pad, cos, sin)

    wk_c = jax.nn.softmax(w_ck)
    wv_c = jax.nn.softmax(w_cv)
    cpos = jnp.arange(C_PAD)[:, None]
    tpos = jnp.arange(T)[None, :]
    dlt = tpos - cpos * STRIDE
    band = (dlt >= 0) & (dlt < KERNEL_W) & (cpos < NUM_C)
    pool_k = jnp.where(band, jnp.take(wk_c, jnp.clip(dlt, 0, KERNEL_W - 1)), 0.0)
    pool_v = jnp.where(band, jnp.take(wv_c, jnp.clip(dlt, 0, KERNEL_W - 1)), 0.0)
    ck, cv = _pool(pool_k.astype(jnp.bfloat16), pool_v.astype(jnp.bfloat16), kh, vh)

    g_pad = jnp.pad(
        gsig[:, : NQ * 3].reshape(T, NQ, 3).transpose(1, 2, 0), ((0, 0), (0, 5), (0, 0))
    )

    out = _attention(qh, kh, vh, ck, cv, g_pad)
    y = _matmul(out, Wo.astype(jnp.bfloat16), bn=256)
    return y.reshape(B, S, H)


# speedup vs baseline: 12.4821x; 11.8229x over previous
import jax
import jax.numpy as jnp
from jax.experimental import pallas as pl
from jax.experimental.pallas import tpu as pltpu

HIDDEN = 2048
NQ = 16
NKV = 4
DH = 128
G = NQ // NKV
KERNEL_W = 32
STRIDE = 16
WIN = 512
THETA = 500000.0
T = 2048
NUM_C = (T - KERNEL_W) // STRIDE + 1
C_PAD = 128
QT = 256
WCOLS = 3 * QT


def _llama3_inv_freq():
    inv = 1.0 / (THETA ** (jnp.arange(0, DH, 2, dtype=jnp.float32) / DH))
    factor, lo, hi, orig = 8.0, 1.0, 4.0, 8192.0
    wavelen = 2.0 * jnp.pi / inv
    smooth = jnp.clip((orig / wavelen - lo) / (hi - lo), 0.0, 1.0)
    return jnp.where(
        wavelen > orig / lo,
        inv / factor,
        jnp.where(wavelen < orig / hi, inv, (1.0 - smooth) * inv / factor + smooth * inv),
    )


def _prep_body(x_ref, wq_ref, wk_ref, wv_ref, wg_ref, cos_ref, sin_ref,
               q_ref, k_ref, v_ref, g_ref):
    xb = x_ref[...].astype(jnp.bfloat16)
    qp = jnp.dot(xb, wq_ref[...], preferred_element_type=jnp.float32)
    kp = jnp.dot(xb, wk_ref[...], preferred_element_type=jnp.float32)
    vp = jnp.dot(xb, wv_ref[...], preferred_element_type=jnp.float32)
    gp = jnp.dot(xb, wg_ref[...], preferred_element_type=jnp.float32)
    g_ref[...] = jax.nn.sigmoid(gp)
    cos = cos_ref[...]
    sin = sin_ref[...]
    scale = DH ** -0.5
    for h in range(NQ):
        x1 = qp[:, h * DH : h * DH + 64]
        x2 = qp[:, h * DH + 64 : (h + 1) * DH]
        r = jnp.concatenate([x1 * cos - x2 * sin, x1 * sin + x2 * cos], axis=1)
        q_ref[h] = (r * scale).astype(jnp.bfloat16)
    for n in range(NKV):
        x1 = kp[:, n * DH : n * DH + 64]
        x2 = kp[:, n * DH + 64 : (n + 1) * DH]
        r = jnp.concatenate([x1 * cos - x2 * sin, x1 * sin + x2 * cos], axis=1)
        k_ref[n] = r.astype(jnp.bfloat16)
        v_ref[n] = vp[:, n * DH : (n + 1) * DH].astype(jnp.bfloat16)


def _prep(x, wq_p, wk_p, wv, wg_pad, cos, sin):
    return pl.pallas_call(
        _prep_body,
        grid=(T // QT,),
        in_specs=[
            pl.BlockSpec((QT, HIDDEN), lambda i: (i, 0)),
            pl.BlockSpec((HIDDEN, NQ * DH), lambda i: (0, 0)),
            pl.BlockSpec((HIDDEN, NKV * DH), lambda i: (0, 0)),
            pl.BlockSpec((HIDDEN, NKV * DH), lambda i: (0, 0)),
            pl.BlockSpec((HIDDEN, 128), lambda i: (0, 0)),
            pl.BlockSpec((QT, 64), lambda i: (i, 0)),
            pl.BlockSpec((QT, 64), lambda i: (i, 0)),
        ],
        out_specs=[
            pl.BlockSpec((NQ, QT, DH), lambda i: (0, i, 0)),
            pl.BlockSpec((NKV, QT, DH), lambda i: (0, i, 0)),
            pl.BlockSpec((NKV, QT, DH), lambda i: (0, i, 0)),
            pl.BlockSpec((QT, 128), lambda i: (i, 0)),
        ],
        out_shape=[
            jax.ShapeDtypeStruct((NQ, T, DH), jnp.bfloat16),
            jax.ShapeDtypeStruct((NKV, T, DH), jnp.bfloat16),
            jax.ShapeDtypeStruct((NKV, T, DH), jnp.bfloat16),
            jax.ShapeDtypeStruct((T, 128), jnp.float32),
        ],
    )(x, wq_p, wk_p, wv, wg_pad, cos, sin)


NT = T // QT


def _attn_body_for(I):
    KW = (I + 1) * QT
    WS = max(I - 2, 0) * QT
    WC = KW - WS

    def body(q_ref, k_ref, v_ref, ck_ref, cv_ref, g_ref, o_ref):
        qb = q_ref[0]
        kb = k_ref[0]
        rows = jax.lax.broadcasted_iota(jnp.int32, (QT, KW), 0) + I * QT
        cols = jax.lax.broadcasted_iota(jnp.int32, (QT, KW), 1)

        s = jax.lax.dot_general(
            qb, kb, (((1,), (1,)), ((), ())), preferred_element_type=jnp.float32
        )
        s = jnp.where(rows >= cols, s, jnp.float32(-1e9))
        mx = jnp.max(s, axis=-1, keepdims=True)
        e = jnp.exp(s - mx)
        l_s = jnp.sum(e, axis=-1, keepdims=True)
        out_s = jnp.dot(
            e.astype(jnp.bfloat16), v_ref[0], preferred_element_type=jnp.float32
        ) / l_s

        ew = e[:, WS:]
        wrows = rows[:, WS:]
        wcols = cols[:, WS:]
        ew = jnp.where((wrows - wcols) < WIN, ew, jnp.float32(0.0))
        l_w = jnp.sum(ew, axis=-1, keepdims=True)
        vw = v_ref[0, WS:KW, :]
        out_w = jnp.dot(
            ew.astype(jnp.bfloat16), vw, preferred_element_type=jnp.float32
        ) / l_w

        ccols = jax.lax.broadcasted_iota(jnp.int32, (QT, C_PAD), 1)
        crows = jax.lax.broadcasted_iota(jnp.int32, (QT, C_PAD), 0) + I * QT
        cmask = (crows >= ccols * STRIDE + KERNEL_W - 1) & (ccols < NUM_C)
        s_c = jax.lax.dot_general(
            qb, ck_ref[0], (((1,), (1,)), ((), ())), preferred_element_type=jnp.float32
        )
        s_c = jnp.where(cmask, s_c, jnp.float32(-1e9))
        mc = jnp.max(s_c, axis=-1, keepdims=True)
        ec = jnp.exp(s_c - mc)
        l_c = jnp.sum(ec, axis=-1, keepdims=True)
        out_c = jnp.dot(
            ec.astype(jnp.bfloat16), cv_ref[0], preferred_element_type=jnp.float32
        )
        if I == 0:
            valid = (crows[:, :1] >= (KERNEL_W - 1)).astype(jnp.float32)
            out_c = out_c * (valid / l_c)
        else:
            out_c = out_c / l_c

        g0 = g_ref[0, 0, :][:, None]
        g1 = g_ref[0, 1, :][:, None]
        g2 = g_ref[0, 2, :][:, None]
        o_ref[...] = (g0 * out_c + g1 * out_s + g2 * out_w).astype(jnp.bfloat16)

    return body


def _attention(q, k, v, ck, cv, g):
    outs = []
    for I in range(NT):
        KW = (I + 1) * QT
        outs.append(
            pl.pallas_call(
                _attn_body_for(I),
                grid=(NQ,),
                in_specs=[
                    pl.BlockSpec((1, QT, DH), lambda h, I=I: (h, I, 0)),
                    pl.BlockSpec((1, KW, DH), lambda h: (h // G, 0, 0)),
                    pl.BlockSpec((1, KW, DH), lambda h: (h // G, 0, 0)),
                    pl.BlockSpec((1, C_PAD, DH), lambda h: (h // G, 0, 0)),
                    pl.BlockSpec((1, C_PAD, DH), lambda h: (h // G, 0, 0)),
                    pl.BlockSpec((1, 8, QT), lambda h, I=I: (h, 0, I)),
                ],
                out_specs=pl.BlockSpec((QT, DH), lambda h: (0, h)),
                out_shape=jax.ShapeDtypeStruct((QT, NQ * DH), jnp.bfloat16),
            )(q, k, v, ck, cv, g)
        )
    return jnp.concatenate(outs, axis=0)


def _pool_body(pk_ref, pv_ref, k_ref, v_ref, ck_ref, cv_ref):
    ck_ref[0] = jnp.dot(pk_ref[...], k_ref[0], preferred_element_type=jnp.float32).astype(
        jnp.bfloat16
    )
    cv_ref[0] = jnp.dot(pv_ref[...], v_ref[0], preferred_element_type=jnp.float32).astype(
        jnp.bfloat16
    )


def _pool(pool_k, pool_v, k, v):
    return pl.pallas_call(
        _pool_body,
        grid=(NKV,),
        in_specs=[
            pl.BlockSpec((C_PAD, T), lambda n: (0, 0)),
            pl.BlockSpec((C_PAD, T), lambda n: (0, 0)),
            pl.BlockSpec((1, T, DH), lambda n: (n, 0, 0)),
            pl.BlockSpec((1, T, DH), lambda n: (n, 0, 0)),
        ],
        out_specs=[
            pl.BlockSpec((1, C_PAD, DH), lambda n: (n, 0, 0)),
            pl.BlockSpec((1, C_PAD, DH), lambda n: (n, 0, 0)),
        ],
        out_shape=[
            jax.ShapeDtypeStruct((NKV, C_PAD, DH), jnp.bfloat16),
            jax.ShapeDtypeStruct((NKV, C_PAD, DH), jnp.bfloat16),
        ],
    )(pool_k, pool_v, k, v)


def _mm_body(x_ref, w_ref, o_ref):
    o_ref[...] = jnp.dot(x_ref[...], w_ref[...], preferred_element_type=jnp.float32)


def _matmul(x, w, bn):
    M, K = x.shape
    _, N = w.shape
    return pl.pallas_call(
        _mm_body,
        grid=(N // bn,),
        in_specs=[
            pl.BlockSpec((M, K), lambda j: (0, 0)),
            pl.BlockSpec((K, bn), lambda j: (0, j)),
        ],
        out_specs=pl.BlockSpec((M, bn), lambda j: (0, j)),
        out_shape=jax.ShapeDtypeStruct((M, N), jnp.float32),
    )(x, w)


def _deinterleave_cols(w, nheads):
    return w.reshape(HIDDEN, nheads, 64, 2).transpose(0, 1, 3, 2).reshape(HIDDEN, nheads * DH)


def kernel(hidden_states, Wq, Wk, Wv, Wo, Wg, w_ck, w_cv):
    B, S, H = hidden_states.shape
    x = hidden_states.reshape(B * S, H)

    wq_p = _deinterleave_cols(Wq, NQ).astype(jnp.bfloat16)
    wk_p = _deinterleave_cols(Wk, NKV).astype(jnp.bfloat16)
    wv_b = Wv.astype(jnp.bfloat16)
    wg_pad = jnp.pad(Wg, ((0, 0), (0, 128 - NQ * 3))).astype(jnp.bfloat16)

    pos = jnp.arange(T, dtype=jnp.float32)
    f = pos[:, None] * _llama3_inv_freq()[None, :]
    cos = jnp.cos(f)
    sin = jnp.sin(f)

    qh, kh, vh, gsig = _prep(x, wq_p, wk_p, wv_b, wg_pad, cos, sin)

    wk_c = jax.nn.softmax(w_ck)
    wv_c = jax.nn.softmax(w_cv)
    cpos = jnp.arange(C_PAD)[:, None]
    tpos = jnp.arange(T)[None, :]
    dlt = tpos - cpos * STRIDE
    live = cpos < NUM_C
    oh = (dlt[None, :, :] == jnp.arange(KERNEL_W)[:, None, None]) & live[None, :, :]
    ohf = oh.astype(jnp.float32)
    pool_k = jnp.einsum("jct,j->ct", ohf, wk_c)
    pool_v = jnp.einsum("jct,j->ct", ohf, wv_c)
    ck, cv = _pool(pool_k.astype(jnp.bfloat16), pool_v.astype(jnp.bfloat16), kh, vh)

    g_pad = jnp.pad(
        gsig[:, : NQ * 3].reshape(T, NQ, 3).transpose(1, 2, 0), ((0, 0), (0, 5), (0, 0))
    )

    out = _attention(qh, kh, vh, ck, cv, g_pad)
    y = _matmul(out, Wo.astype(jnp.bfloat16), bn=256)
    return y.reshape(B, S, H)


# speedup vs baseline: 13.0752x; 1.0475x over previous
import jax
import jax.numpy as jnp
from jax.experimental import pallas as pl
from jax.experimental.pallas import tpu as pltpu

HIDDEN = 2048
NQ = 16
NKV = 4
DH = 128
G = NQ // NKV
KERNEL_W = 32
STRIDE = 16
WIN = 512
THETA = 500000.0
T = 2048
NUM_C = (T - KERNEL_W) // STRIDE + 1
C_PAD = 128
QT = 256
WCOLS = 3 * QT


def _llama3_inv_freq():
    inv = 1.0 / (THETA ** (jnp.arange(0, DH, 2, dtype=jnp.float32) / DH))
    factor, lo, hi, orig = 8.0, 1.0, 4.0, 8192.0
    wavelen = 2.0 * jnp.pi / inv
    smooth = jnp.clip((orig / wavelen - lo) / (hi - lo), 0.0, 1.0)
    return jnp.where(
        wavelen > orig / lo,
        inv / factor,
        jnp.where(wavelen < orig / hi, inv, (1.0 - smooth) * inv / factor + smooth * inv),
    )


def _prep_body(x_ref, wq_ref, wk_ref, wv_ref, wg_ref, cos_ref, sin_ref, sw_ref,
               q_ref, k_ref, v_ref, g_ref):
    xb = x_ref[...].astype(jnp.bfloat16)
    qp = jnp.dot(xb, wq_ref[...], preferred_element_type=jnp.float32)
    kp = jnp.dot(xb, wk_ref[...], preferred_element_type=jnp.float32)
    vp = jnp.dot(xb, wv_ref[...], preferred_element_type=jnp.float32)
    gp = jnp.dot(xb, wg_ref[...], preferred_element_type=jnp.float32)
    g_ref[...] = jax.nn.sigmoid(gp)
    cos = cos_ref[...]
    sin = sin_ref[...]
    sw = sw_ref[...]
    scale = DH ** -0.5
    for h in range(NQ):
        xh = qp[:, h * DH : (h + 1) * DH]
        xs = jnp.dot(xh.astype(jnp.bfloat16), sw, preferred_element_type=jnp.float32)
        q_ref[h] = ((xh * cos + xs * sin) * scale).astype(jnp.bfloat16)
    for n in range(NKV):
        xh = kp[:, n * DH : (n + 1) * DH]
        xs = jnp.dot(xh.astype(jnp.bfloat16), sw, preferred_element_type=jnp.float32)
        k_ref[n] = (xh * cos + xs * sin).astype(jnp.bfloat16)
        v_ref[n] = vp[:, n * DH : (n + 1) * DH].astype(jnp.bfloat16)


def _prep(x, wq, wk, wv, wg_pad, cos, sin, sw):
    return pl.pallas_call(
        _prep_body,
        grid=(T // QT,),
        in_specs=[
            pl.BlockSpec((QT, HIDDEN), lambda i: (i, 0)),
            pl.BlockSpec((HIDDEN, NQ * DH), lambda i: (0, 0)),
            pl.BlockSpec((HIDDEN, NKV * DH), lambda i: (0, 0)),
            pl.BlockSpec((HIDDEN, NKV * DH), lambda i: (0, 0)),
            pl.BlockSpec((HIDDEN, 128), lambda i: (0, 0)),
            pl.BlockSpec((QT, DH), lambda i: (i, 0)),
            pl.BlockSpec((QT, DH), lambda i: (i, 0)),
            pl.BlockSpec((DH, DH), lambda i: (0, 0)),
        ],
        out_specs=[
            pl.BlockSpec((NQ, QT, DH), lambda i: (0, i, 0)),
            pl.BlockSpec((NKV, QT, DH), lambda i: (0, i, 0)),
            pl.BlockSpec((NKV, QT, DH), lambda i: (0, i, 0)),
            pl.BlockSpec((QT, 128), lambda i: (i, 0)),
        ],
        out_shape=[
            jax.ShapeDtypeStruct((NQ, T, DH), jnp.bfloat16),
            jax.ShapeDtypeStruct((NKV, T, DH), jnp.bfloat16),
            jax.ShapeDtypeStruct((NKV, T, DH), jnp.bfloat16),
            jax.ShapeDtypeStruct((T, 128), jnp.float32),
        ],
    )(x, wq, wk, wv, wg_pad, cos, sin, sw)


NT = T // QT


def _attn_body_for(I):
    KW = (I + 1) * QT
    WS = max(I - 2, 0) * QT
    WC = KW - WS

    def body(q_ref, k_ref, v_ref, ck_ref, cv_ref, g_ref, sel_ref, o_ref):
        qb = q_ref[0]
        kb = k_ref[0]
        rows = jax.lax.broadcasted_iota(jnp.int32, (QT, KW), 0) + I * QT
        cols = jax.lax.broadcasted_iota(jnp.int32, (QT, KW), 1)

        s = jax.lax.dot_general(
            qb, kb, (((1,), (1,)), ((), ())), preferred_element_type=jnp.float32
        )
        s = jnp.where(rows >= cols, s, jnp.float32(-1e9))
        mx = jnp.max(s, axis=-1, keepdims=True)
        e = jnp.exp(s - mx)
        l_s = jnp.sum(e, axis=-1, keepdims=True)
        out_s = jnp.dot(
            e.astype(jnp.bfloat16), v_ref[0], preferred_element_type=jnp.float32
        ) / l_s

        ew = e[:, WS:]
        wrows = rows[:, WS:]
        wcols = cols[:, WS:]
        ew = jnp.where((wrows - wcols) < WIN, ew, jnp.float32(0.0))
        l_w = jnp.sum(ew, axis=-1, keepdims=True)
        vw = v_ref[0, WS:KW, :]
        out_w = jnp.dot(
            ew.astype(jnp.bfloat16), vw, preferred_element_type=jnp.float32
        ) / l_w

        ccols = jax.lax.broadcasted_iota(jnp.int32, (QT, C_PAD), 1)
        crows = jax.lax.broadcasted_iota(jnp.int32, (QT, C_PAD), 0) + I * QT
        cmask = (crows >= ccols * STRIDE + KERNEL_W - 1) & (ccols < NUM_C)
        s_c = jax.lax.dot_general(
            qb, ck_ref[0], (((1,), (1,)), ((), ())), preferred_element_type=jnp.float32
        )
        s_c = jnp.where(cmask, s_c, jnp.float32(-1e9))
        mc = jnp.max(s_c, axis=-1, keepdims=True)
        ec = jnp.exp(s_c - mc)
        l_c = jnp.sum(ec, axis=-1, keepdims=True)
        out_c = jnp.dot(
            ec.astype(jnp.bfloat16), cv_ref[0], preferred_element_type=jnp.float32
        )
        if I == 0:
            valid = (crows[:, :1] >= (KERNEL_W - 1)).astype(jnp.float32)
            out_c = out_c * (valid / l_c)
        else:
            out_c = out_c / l_c

        gsel = jnp.dot(
            g_ref[...].astype(jnp.bfloat16), sel_ref[0], preferred_element_type=jnp.float32
        )
        g0 = gsel[:, 0:1]
        g1 = gsel[:, 1:2]
        g2 = gsel[:, 2:3]
        o_ref[...] = (g0 * out_c + g1 * out_s + g2 * out_w).astype(jnp.bfloat16)

    return body


def _attention(q, k, v, ck, cv, gsig, sel_g):
    outs = []
    for I in range(NT):
        KW = (I + 1) * QT
        outs.append(
            pl.pallas_call(
                _attn_body_for(I),
                grid=(NQ,),
                in_specs=[
                    pl.BlockSpec((1, QT, DH), lambda h, I=I: (h, I, 0)),
                    pl.BlockSpec((1, KW, DH), lambda h: (h // G, 0, 0)),
                    pl.BlockSpec((1, KW, DH), lambda h: (h // G, 0, 0)),
                    pl.BlockSpec((1, C_PAD, DH), lambda h: (h // G, 0, 0)),
                    pl.BlockSpec((1, C_PAD, DH), lambda h: (h // G, 0, 0)),
                    pl.BlockSpec((QT, 128), lambda h, I=I: (I, 0)),
                    pl.BlockSpec((1, 128, 128), lambda h: (h, 0, 0)),
                ],
                out_specs=pl.BlockSpec((QT, DH), lambda h: (0, h)),
                out_shape=jax.ShapeDtypeStruct((QT, NQ * DH), jnp.bfloat16),
            )(q, k, v, ck, cv, gsig, sel_g)
        )
    return jnp.concatenate(outs, axis=0)


def _pool_body(pk_ref, pv_ref, k_ref, v_ref, ck_ref, cv_ref):
    ck_ref[0] = jnp.dot(pk_ref[...], k_ref[0], preferred_element_type=jnp.float32).astype(
        jnp.bfloat16
    )
    cv_ref[0] = jnp.dot(pv_ref[...], v_ref[0], preferred_element_type=jnp.float32).astype(
        jnp.bfloat16
    )


def _pool(pool_k, pool_v, k, v):
    return pl.pallas_call(
        _pool_body,
        grid=(NKV,),
        in_specs=[
            pl.BlockSpec((C_PAD, T), lambda n: (0, 0)),
            pl.BlockSpec((C_PAD, T), lambda n: (0, 0)),
            pl.BlockSpec((1, T, DH), lambda n: (n, 0, 0)),
            pl.BlockSpec((1, T, DH), lambda n: (n, 0, 0)),
        ],
        out_specs=[
            pl.BlockSpec((1, C_PAD, DH), lambda n: (n, 0, 0)),
            pl.BlockSpec((1, C_PAD, DH), lambda n: (n, 0, 0)),
        ],
        out_shape=[
            jax.ShapeDtypeStruct((NKV, C_PAD, DH), jnp.bfloat16),
            jax.ShapeDtypeStruct((NKV, C_PAD, DH), jnp.bfloat16),
        ],
    )(pool_k, pool_v, k, v)


def _mm_body(x_ref, w_ref, o_ref):
    o_ref[...] = jnp.dot(x_ref[...], w_ref[...], preferred_element_type=jnp.float32)


def _matmul(x, w, bn):
    M, K = x.shape
    _, N = w.shape
    return pl.pallas_call(
        _mm_body,
        grid=(N // bn,),
        in_specs=[
            pl.BlockSpec((M, K), lambda j: (0, 0)),
            pl.BlockSpec((K, bn), lambda j: (0, j)),
        ],
        out_specs=pl.BlockSpec((M, bn), lambda j: (0, j)),
        out_shape=jax.ShapeDtypeStruct((M, N), jnp.float32),
    )(x, w)


def kernel(hidden_states, Wq, Wk, Wv, Wo, Wg, w_ck, w_cv):
    B, S, H = hidden_states.shape
    x = hidden_states.reshape(B * S, H)

    wq_b = Wq.astype(jnp.bfloat16)
    wk_b = Wk.astype(jnp.bfloat16)
    wv_b = Wv.astype(jnp.bfloat16)
    wg_pad = jnp.pad(Wg, ((0, 0), (0, 128 - NQ * 3))).astype(jnp.bfloat16)

    pos = jnp.arange(T, dtype=jnp.float32)
    f = pos[:, None] * _llama3_inv_freq()[None, :]
    cos = jnp.repeat(jnp.cos(f), 2, axis=1)
    sin = jnp.repeat(jnp.sin(f), 2, axis=1)
    r_ = jnp.arange(DH)[:, None]
    c_ = jnp.arange(DH)[None, :]
    sw = (
        jnp.where((r_ == c_ + 1) & (c_ % 2 == 0), -1.0, 0.0)
        + jnp.where((c_ == r_ + 1) & (r_ % 2 == 0), 1.0, 0.0)
    ).astype(jnp.bfloat16)

    qh, kh, vh, gsig = _prep(x, wq_b, wk_b, wv_b, wg_pad, cos, sin, sw)

    wk_c = jax.nn.softmax(w_ck)
    wv_c = jax.nn.softmax(w_cv)
    cpos = jnp.arange(C_PAD)[:, None]
    tpos = jnp.arange(T)[None, :]
    dlt = tpos - cpos * STRIDE
    live = cpos < NUM_C
    oh = (dlt[None, :, :] == jnp.arange(KERNEL_W)[:, None, None]) & live[None, :, :]
    ohf = oh.astype(jnp.float32)
    pool_k = jnp.einsum("jct,j->ct", ohf, wk_c)
    pool_v = jnp.einsum("jct,j->ct", ohf, wv_c)
    ck, cv = _pool(pool_k.astype(jnp.bfloat16), pool_v.astype(jnp.bfloat16), kh, vh)

    h_ = jnp.arange(NQ)[:, None, None]
    rr = jnp.arange(128)[None, :, None]
    cc = jnp.arange(128)[None, None, :]
    sel_g = ((rr == 3 * h_ + cc) & (cc < 3)).astype(jnp.bfloat16)

    out = _attention(qh, kh, vh, ck, cv, gsig, sel_g)
    y = _matmul(out, Wo.astype(jnp.bfloat16), bn=256)
    return y.reshape(B, S, H)
